# speedup vs baseline: 6.5942x; 6.5942x over previous
import functools

import jax
import jax.numpy as jnp
from jax import lax
from jax.experimental import pallas as pl
from jax.experimental.pallas import tpu as pltpu
from jax.experimental.pallas import tpu_sc as plsc

_D = 128
_L = 128
_SUB = 2
_C = _SUB * _L
_NW = 32


def _emb_body(ids_hbm, table_hbm, out_hbm, idx_v, rows_v,
              gsem0, gsem1, wsem0, wsem1, *, nchunk, per_w):
  wid = lax.axis_index("s") * 2 + lax.axis_index("c")
  row_base = wid * (per_w // _L)
  base = wid * per_w

  gsems = (gsem0, gsem1)
  wsems = (wsem0, wsem1)

  def idx_load(c, s):
    pltpu.sync_copy(ids_hbm.at[pl.ds(row_base + c * _SUB, _SUB)], idx_v.at[s])

  def gather_start(s):
    for j in range(_SUB):
      pltpu.make_async_copy(
          table_hbm.at[idx_v.at[s, j]],
          rows_v.at[s].at[pl.ds(j * _L, _L)],
          gsems[s]).start()

  def gather_wait(s):
    for j in range(_SUB):
      pltpu.make_async_copy(
          table_hbm.at[idx_v.at[s, j]],
          rows_v.at[s].at[pl.ds(j * _L, _L)],
          gsems[s]).wait()

  def write_start(c, s):
    pltpu.make_async_copy(
        rows_v.at[s], out_hbm.at[pl.ds(base + c * _C, _C)], wsems[s]).start()

  def write_wait(c, s):
    pltpu.make_async_copy(
        rows_v.at[s], out_hbm.at[pl.ds(base + c * _C, _C)], wsems[s]).wait()

  idx_load(0, 0)
  gather_start(0)
  idx_load(1, 1)
  gather_start(1)
  gather_wait(0)
  write_start(0, 0)

  def body(g, carry):
    i0 = 2 * g
    write_wait(i0 - 2, 0)
    idx_load(i0, 0)
    gather_start(0)
    gather_wait(1)
    write_start(i0 - 1, 1)
    write_wait(i0 - 1, 1)
    idx_load(i0 + 1, 1)
    gather_start(1)
    gather_wait(0)
    write_start(i0, 0)
    return carry

  lax.fori_loop(1, nchunk // 2, body, 0)

  gather_wait(1)
  write_start(nchunk - 1, 1)
  write_wait(nchunk - 2, 0)
  write_wait(nchunk - 1, 1)


def kernel(word_ids, n_words, table):
  del n_words
  b, h = word_ids.shape
  n = b * h
  ids = word_ids.reshape(n // _L, _L).astype(jnp.int32)
  per_w = n // _NW
  nchunk = per_w // _C
  mesh = plsc.VectorSubcoreMesh(core_axis_name="c", subcore_axis_name="s")
  out = pl.kernel(
      functools.partial(_emb_body, nchunk=nchunk, per_w=per_w),
      out_type=jax.ShapeDtypeStruct((n, _D), table.dtype),
      mesh=mesh,
      scratch_types=[
          pltpu.VMEM((2, _SUB, _L), jnp.int32),
          pltpu.VMEM((2, _C, _D), jnp.float32),
          pltpu.SemaphoreType.DMA,
          pltpu.SemaphoreType.DMA,
          pltpu.SemaphoreType.DMA,
          pltpu.SemaphoreType.DMA,
      ],
  )(ids, table)
  return out.reshape(b, h, _D)


# speedup vs baseline: 13.0418x; 1.9778x over previous
import functools

import jax
import jax.numpy as jnp
from jax import lax
from jax.experimental import pallas as pl
from jax.experimental.pallas import tpu as pltpu
from jax.experimental.pallas import tpu_sc as plsc

_D = 128
_L = 128
_SUB = 2
_C = _SUB * _L
_NW = 32


def _emb_body(ids_hbm, table_hbm, out_hbm, idx_v, rows_v, table_sp,
              gsem0, gsem1, wsem0, wsem1, *, nchunk, per_w):
  sid = lax.axis_index("s")
  wid = sid * 2 + lax.axis_index("c")
  row_base = wid * (per_w // _L)
  base = wid * per_w

  @pl.when(sid == 0)
  def _stage():
    pltpu.sync_copy(table_hbm, table_sp)

  plsc.subcore_barrier()

  gsems = (gsem0, gsem1)
  wsems = (wsem0, wsem1)

  def idx_load(c, s):
    pltpu.sync_copy(ids_hbm.at[pl.ds(row_base + c * _SUB, _SUB)], idx_v.at[s])

  def gather_start(s):
    for j in range(_SUB):
      pltpu.make_async_copy(
          table_sp.at[idx_v.at[s, j]],
          rows_v.at[s].at[pl.ds(j * _L, _L)],
          gsems[s]).start()

  def gather_wait(s):
    for j in range(_SUB):
      pltpu.make_async_copy(
          table_sp.at[idx_v.at[s, j]],
          rows_v.at[s].at[pl.ds(j * _L, _L)],
          gsems[s]).wait()

  def write_start(c, s):
    pltpu.make_async_copy(
        rows_v.at[s], out_hbm.at[pl.ds(base + c * _C, _C)], wsems[s]).start()

  def write_wait(c, s):
    pltpu.make_async_copy(
        rows_v.at[s], out_hbm.at[pl.ds(base + c * _C, _C)], wsems[s]).wait()

  idx_load(0, 0)
  gather_start(0)
  idx_load(1, 1)
  gather_start(1)
  gather_wait(0)
  write_start(0, 0)

  def body(g, carry):
    i0 = 2 * g
    write_wait(i0 - 2, 0)
    idx_load(i0, 0)
    gather_start(0)
    gather_wait(1)
    write_start(i0 - 1, 1)
    write_wait(i0 - 1, 1)
    idx_load(i0 + 1, 1)
    gather_start(1)
    gather_wait(0)
    write_start(i0, 0)
    return carry

  lax.fori_loop(1, nchunk // 2, body, 0)

  gather_wait(1)
  write_start(nchunk - 1, 1)
  write_wait(nchunk - 2, 0)
  write_wait(nchunk - 1, 1)


def kernel(word_ids, n_words, table):
  del n_words
  b, h = word_ids.shape
  n = b * h
  ids = word_ids.reshape(n // _L, _L).astype(jnp.int32)
  per_w = n // _NW
  nchunk = per_w // _C
  mesh = plsc.VectorSubcoreMesh(core_axis_name="c", subcore_axis_name="s")
  out = pl.kernel(
      functools.partial(_emb_body, nchunk=nchunk, per_w=per_w),
      out_type=jax.ShapeDtypeStruct((n, _D), table.dtype),
      mesh=mesh,
      scratch_types=[
          pltpu.VMEM((2, _SUB, _L), jnp.int32),
          pltpu.VMEM((2, _C, _D), jnp.float32),
          pltpu.VMEM_SHARED((table.shape[0], _D), jnp.float32),
          pltpu.SemaphoreType.DMA,
          pltpu.SemaphoreType.DMA,
          pltpu.SemaphoreType.DMA,
          pltpu.SemaphoreType.DMA,
      ],
  )(ids, table)
  return out.reshape(b, h, _D)


# speedup vs baseline: 15.4198x; 1.1823x over previous
import functools

import jax
import jax.numpy as jnp
from jax import lax
from jax.experimental import pallas as pl
from jax.experimental.pallas import tpu as pltpu
from jax.experimental.pallas import tpu_sc as plsc

_D = 128
_L = 128
_SUB = 2
_C = _SUB * _L
_NS = 3
_NW = 32


def _emb_body(ids_hbm, table_hbm, out_hbm, idx_v, rows_v, table_sp,
              gsem0, gsem1, gsem2, wsem0, wsem1, wsem2, *, nchunk, per_w):
  sid = lax.axis_index("s")
  wid = sid * 2 + lax.axis_index("c")
  base = wid * per_w

  @pl.when(sid == 0)
  def _stage():
    pltpu.sync_copy(table_hbm, table_sp)

  plsc.subcore_barrier()

  row_base = wid * (per_w // _L)
  gsems = (gsem0, gsem1, gsem2)
  wsems = (wsem0, wsem1, wsem2)

  def idx_load(c, s):
    pltpu.sync_copy(ids_hbm.at[pl.ds(row_base + c * _SUB, _SUB)], idx_v.at[s])

  def gather_start(c, s):
    del c
    for j in range(_SUB):
      pltpu.make_async_copy(
          table_sp.at[idx_v.at[s, j]],
          rows_v.at[s].at[pl.ds(j * _L, _L)],
          gsems[s]).start()

  def gather_wait(c, s):
    del c
    for j in range(_SUB):
      pltpu.make_async_copy(
          table_sp.at[idx_v.at[s, j]],
          rows_v.at[s].at[pl.ds(j * _L, _L)],
          gsems[s]).wait()

  def write_start(c, s):
    pltpu.make_async_copy(
        rows_v.at[s], out_hbm.at[pl.ds(base + c * _C, _C)], wsems[s]).start()

  def write_wait(c, s):
    pltpu.make_async_copy(
        rows_v.at[s], out_hbm.at[pl.ds(base + c * _C, _C)], wsems[s]).wait()

  for c in range(_NS):
    idx_load(c, c)
    gather_start(c, c)
  gather_wait(0, 0)
  write_start(0, 0)
  gather_wait(1, 1)
  write_start(1, 1)

  def body(g, carry):
    for d in range(_NS):
      i = _NS * g + d
      write_wait(i - _NS, d)
      idx_load(i, d)
      gather_start(i, d)
      gather_wait(i - 1, (d - 1) % _NS)
      write_start(i - 1, (d - 1) % _NS)
    return carry

  lax.fori_loop(1, (nchunk - 1) // _NS, body, 0)

  last = nchunk - 1
  write_wait(last - _NS, 0)
  idx_load(last, 0)
  gather_start(last, 0)
  gather_wait(last - 1, 2)
  write_start(last - 1, 2)
  gather_wait(last, 0)
  write_start(last, 0)
  write_wait(last - 2, 1)
  write_wait(last - 1, 2)
  write_wait(last, 0)


def kernel(word_ids, n_words, table):
  del n_words
  b, h = word_ids.shape
  n = b * h
  ids = word_ids.reshape(n // _L, _L).astype(jnp.int32)
  per_w = n // _NW
  nchunk = per_w // _C
  mesh = plsc.VectorSubcoreMesh(core_axis_name="c", subcore_axis_name="s")
  out = pl.kernel(
      functools.partial(_emb_body, nchunk=nchunk, per_w=per_w),
      out_type=jax.ShapeDtypeStruct((n, _D), table.dtype),
      mesh=mesh,
      scratch_types=[
          pltpu.VMEM((_NS, _SUB, _L), jnp.int32),
          pltpu.VMEM((_NS, _C, _D), jnp.float32),
          pltpu.VMEM_SHARED((table.shape[0], _D), jnp.float32),
          pltpu.SemaphoreType.DMA,
          pltpu.SemaphoreType.DMA,
          pltpu.SemaphoreType.DMA,
          pltpu.SemaphoreType.DMA,
          pltpu.SemaphoreType.DMA,
          pltpu.SemaphoreType.DMA,
      ],
  )(ids, table)
  return out.reshape(b, h, _D)
